# Initial kernel scaffold; baseline (speedup 1.0000x reference)
import jax
import jax.numpy as jnp
from jax.experimental import pallas as pl


def kernel(pred_bboxes, pred_logits, gt_bboxes_3d, gt_labels_3d):
    raise NotImplementedError("write your pallas kernel here")


# speedup vs baseline: 13.5710x; 13.5710x over previous
import jax
import jax.numpy as jnp
from jax import lax
from jax.experimental import pallas as pl
from jax.experimental.pallas import tpu as pltpu

B, Q, G, C = 4, 900, 64, 10
CODE_W = [1.0, 1.0, 1.0, 1.0, 1.0, 1.0, 1.0, 1.0, 0.2, 0.2]
ALPHA, GAMMA = 0.25, 2.0
CLS_W, BBOX_W = 2.0, 0.25
EPS = 1e-12
INF = 1e9
BIGI = 2**30


def _body(pb_ref, lg_ref, gtn_ref, glc_ref, aug_ref, out_ref,
          c_ref, cm_ref, fc_ref, rm_ref, asg_ref):
    pb = pb_ref[0]
    lg = lg_ref[0]
    gtn = gtn_ref[0]
    glc = glc_ref[0]
    aug = aug_ref[0]

    p = 1.0 / (1.0 + jnp.exp(-lg))
    omp = 1.0 - p
    posc = -jnp.log(p + EPS) * ALPHA * (omp * omp)
    negc = -jnp.log(omp + EPS) * (1.0 - ALPHA) * (p * p)
    diff = (posc - negc) * CLS_W
    iota_c = lax.broadcasted_iota(jnp.int32, (G, C), 1)
    onehot_lab = (glc == iota_c).astype(jnp.float32)
    c_cls = jnp.dot(onehot_lab, diff, preferred_element_type=jnp.float32)

    n0 = gtn[:, 0:1]
    n1 = gtn[:, 1:2]
    n2 = jnp.log(jnp.clip(gtn[:, 3:4], 1e-6))
    n3 = jnp.log(jnp.clip(gtn[:, 4:5], 1e-6))
    n4 = gtn[:, 2:3]
    n5 = jnp.log(jnp.clip(gtn[:, 5:6], 1e-6))
    n6 = jnp.sin(gtn[:, 6:7])
    n7 = jnp.cos(gtn[:, 6:7])
    reg = jnp.abs(pb[0:1, :] - n0)
    for d, nd in enumerate((n1, n2, n3, n4, n5, n6, n7), start=1):
        reg = reg + jnp.abs(pb[d:d + 1, :] - nd)
    c_ref[...] = c_cls + BBOX_W * reg

    cmat = c_ref[...]
    iota_g = lax.broadcasted_iota(jnp.int32, (G, Q), 0)
    iota_q2 = lax.broadcasted_iota(jnp.int32, (G, Q), 1)
    flat = iota_q2 * G + iota_g
    cm0 = jnp.min(cmat, axis=1, keepdims=True)
    fc0 = jnp.min(jnp.where(cmat == cm0, flat, BIGI), axis=1, keepdims=True)
    cm_ref[...] = cm0
    fc_ref[...] = fc0
    rm_ref[...] = jnp.zeros((1, Q), jnp.float32)
    asg_ref[...] = jnp.zeros((1, Q), jnp.int32)

    iq_row = lax.broadcasted_iota(jnp.int32, (1, Q), 1)
    ig_col = lax.broadcasted_iota(jnp.int32, (G, 1), 0)

    def step(_, carry):
        cm = cm_ref[...]
        fcv = fc_ref[...]
        mv = jnp.min(cm)
        idx = jnp.min(jnp.where(cm == mv, fcv, BIGI))
        qs = idx // G
        gs = idx - qs * G
        rm_ref[...] = jnp.where(iq_row == qs, INF, rm_ref[...])
        asg_ref[...] = jnp.where(iq_row == qs, gs + 1, asg_ref[...])
        cm = jnp.where(ig_col == gs, INF, cm)
        cm_ref[...] = cm
        stale = jnp.logical_and(fcv // G == qs, cm < 1e8)

        @pl.when(jnp.any(stale))
        def _recompute():
            m = c_ref[...] + rm_ref[...]
            cmn = jnp.min(m, axis=1, keepdims=True)
            fcn = jnp.min(jnp.where(m == cmn, flat, BIGI), axis=1, keepdims=True)
            cm_ref[...] = jnp.where(stale, cmn, cm_ref[...])
            fc_ref[...] = jnp.where(stale, fcn, fc_ref[...])

        return carry

    lax.fori_loop(0, G, step, 0)

    asg = asg_ref[...]
    pos = asg > 0
    onehot_qg = (ig_col == (asg - 1)).astype(jnp.float32)
    gath = jnp.dot(aug, onehot_qg, preferred_element_type=jnp.float32)
    labels = jnp.where(pos, gath[9:10, :], float(C))

    iota_cq = lax.broadcasted_iota(jnp.int32, (C, Q), 0)
    t = (iota_cq == labels.astype(jnp.int32)).astype(jnp.float32)
    x = lg
    ce = jnp.maximum(x, 0.0) - x * t + jnp.log(1.0 + jnp.exp(-jnp.abs(x)))
    pt = p * t + omp * (1.0 - t)
    ompt = 1.0 - pt
    fw = (ALPHA * t + (1.0 - ALPHA) * (1.0 - t)) * (ompt * ompt)
    lcls = jnp.sum(ce * fw) * (CLS_W / float(B * G))

    t6 = gath[6:7]
    nt = (gath[0:1], gath[1:2],
          jnp.log(jnp.clip(gath[3:4], 1e-6)),
          jnp.log(jnp.clip(gath[4:5], 1e-6)),
          gath[2:3],
          jnp.log(jnp.clip(gath[5:6], 1e-6)),
          jnp.sin(t6), jnp.cos(t6),
          gath[7:8], gath[8:9])
    finite = jnp.isfinite(nt[0])
    for d in range(1, 10):
        finite = jnp.logical_and(finite, jnp.isfinite(nt[d]))
    bw = pos.astype(jnp.float32) * finite.astype(jnp.float32)
    lbb = jnp.zeros((), jnp.float32)
    for d in range(10):
        sd = jnp.where(finite, nt[d], 0.0)
        lbb = lbb + jnp.sum(jnp.abs(pb[d:d + 1, :] - sd) * bw) * CODE_W[d]
    lbb = lbb * (BBOX_W / float(B * G))

    out_ref[0, 0, 0] = lcls + lbb


_GRID_CALL = pl.pallas_call(
    _body,
    grid=(B,),
    in_specs=[
        pl.BlockSpec((1, 10, Q), lambda b: (b, 0, 0)),
        pl.BlockSpec((1, C, Q), lambda b: (b, 0, 0)),
        pl.BlockSpec((1, G, 9), lambda b: (b, 0, 0)),
        pl.BlockSpec((1, G, 1), lambda b: (b, 0, 0)),
        pl.BlockSpec((1, 10, G), lambda b: (b, 0, 0)),
    ],
    out_specs=pl.BlockSpec((1, 1, 1), lambda b: (b, 0, 0), memory_space=pltpu.SMEM),
    out_shape=jax.ShapeDtypeStruct((B, 1, 1), jnp.float32),
    scratch_shapes=[
        pltpu.VMEM((G, Q), jnp.float32),
        pltpu.VMEM((G, 1), jnp.float32),
        pltpu.VMEM((G, 1), jnp.int32),
        pltpu.VMEM((1, Q), jnp.float32),
        pltpu.VMEM((1, Q), jnp.int32),
    ],
)


def kernel(pred_bboxes, pred_logits, gt_bboxes_3d, gt_labels_3d):
    pb_t = jnp.transpose(pred_bboxes, (0, 2, 1))
    lg_t = jnp.transpose(pred_logits, (0, 2, 1))
    glc = gt_labels_3d[..., None]
    aug = jnp.concatenate(
        [jnp.transpose(gt_bboxes_3d, (0, 2, 1)),
         gt_labels_3d[:, None, :].astype(jnp.float32)], axis=1)
    out = _GRID_CALL(pb_t, lg_t, gt_bboxes_3d, glc, aug)
    return jnp.sum(out[:, 0, 0])


# speedup vs baseline: 25.3187x; 1.8656x over previous
import jax
import jax.numpy as jnp
from jax import lax
from jax.experimental import pallas as pl
from jax.experimental.pallas import tpu as pltpu

B, Q, G, C = 4, 900, 64, 10
CODE_W = [1.0, 1.0, 1.0, 1.0, 1.0, 1.0, 1.0, 1.0, 0.2, 0.2]
ALPHA, GAMMA = 0.25, 2.0
CLS_W, BBOX_W = 2.0, 0.25
EPS = 1e-12
INF = 1e9
BIGI = 2**30


def _body(pb_ref, lg_ref, gtn_ref, glc_ref, aug_ref, out_ref,
          c_ref, cm_ref, fc_ref, rm_ref, asg_ref, gath_ref):
    lg3 = lg_ref[...]
    pb3 = pb_ref[...]

    p3 = 1.0 / (1.0 + jnp.exp(-lg3))
    omp3 = 1.0 - p3
    posc = -jnp.log(p3 + EPS) * ALPHA * (omp3 * omp3)
    negc = -jnp.log(omp3 + EPS) * (1.0 - ALPHA) * (p3 * p3)
    diff3 = (posc - negc) * CLS_W

    iota_c = lax.broadcasted_iota(jnp.int32, (G, C), 1)
    for b in range(B):
        glc = glc_ref[b]
        onehot_lab = (glc == iota_c).astype(jnp.float32)
        c_cls = jnp.dot(onehot_lab, diff3[b], preferred_element_type=jnp.float32)
        gtn = gtn_ref[b]
        n0 = gtn[:, 0:1]
        n1 = gtn[:, 1:2]
        n2 = jnp.log(jnp.clip(gtn[:, 3:4], 1e-6))
        n3 = jnp.log(jnp.clip(gtn[:, 4:5], 1e-6))
        n4 = gtn[:, 2:3]
        n5 = jnp.log(jnp.clip(gtn[:, 5:6], 1e-6))
        n6 = jnp.sin(gtn[:, 6:7])
        n7 = jnp.cos(gtn[:, 6:7])
        pb = pb3[b]
        reg = jnp.abs(pb[0:1, :] - n0)
        for d, nd in enumerate((n1, n2, n3, n4, n5, n6, n7), start=1):
            reg = reg + jnp.abs(pb[d:d + 1, :] - nd)
        c_ref[b] = c_cls + BBOX_W * reg

    flat2 = (lax.broadcasted_iota(jnp.int32, (G, Q), 1) * G
             + lax.broadcasted_iota(jnp.int32, (G, Q), 0))
    c3 = c_ref[...]
    cm0 = jnp.min(c3, axis=2, keepdims=True)
    fc0 = jnp.min(jnp.where(c3 == cm0, flat2[None], BIGI), axis=2, keepdims=True)
    cm_ref[...] = cm0
    fc_ref[...] = fc0
    rm_ref[...] = jnp.zeros((B, 1, Q), jnp.float32)
    asg_ref[...] = jnp.zeros((B, 1, Q), jnp.int32)

    iq3 = lax.broadcasted_iota(jnp.int32, (1, 1, Q), 2)
    ig3 = lax.broadcasted_iota(jnp.int32, (1, G, 1), 1)

    def step(_, carry):
        cm = cm_ref[...]
        fcv = fc_ref[...]
        mv = jnp.min(cm, axis=1, keepdims=True)
        idx = jnp.min(jnp.where(cm == mv, fcv, BIGI), axis=1, keepdims=True)
        qs = idx // G
        gs = idx - qs * G
        rm_ref[...] = jnp.where(iq3 == qs, INF, rm_ref[...])
        asg_ref[...] = jnp.where(iq3 == qs, gs + 1, asg_ref[...])
        cm = jnp.where(ig3 == gs, INF, cm)
        cm_ref[...] = cm
        stale = jnp.logical_and(fcv // G == qs, cm < 1e8)
        for b in range(B):
            @pl.when(jnp.any(stale[b]))
            def _recompute(b=b):
                m = c_ref[b] + rm_ref[b]
                cmn = jnp.min(m, axis=1, keepdims=True)
                fcn = jnp.min(jnp.where(m == cmn, flat2, BIGI),
                              axis=1, keepdims=True)
                cm_ref[b] = jnp.where(stale[b], cmn, cm_ref[b])
                fc_ref[b] = jnp.where(stale[b], fcn, fc_ref[b])
        return carry

    lax.fori_loop(0, G, step, 0)

    asg3 = asg_ref[...]
    onehot3 = (ig3 == (asg3 - 1)).astype(jnp.float32)
    for b in range(B):
        gath_ref[b] = jnp.dot(aug_ref[b], onehot3[b],
                              preferred_element_type=jnp.float32)
    gath3 = gath_ref[...]
    pos3 = asg3 > 0
    labels3 = jnp.where(pos3, gath3[:, 9:10, :], float(C))

    iota_cq = lax.broadcasted_iota(jnp.int32, (1, C, 1), 1)
    t3 = (iota_cq == labels3.astype(jnp.int32)).astype(jnp.float32)
    x3 = lg3
    ce = jnp.maximum(x3, 0.0) - x3 * t3 + jnp.log(1.0 + jnp.exp(-jnp.abs(x3)))
    pt = p3 * t3 + omp3 * (1.0 - t3)
    ompt = 1.0 - pt
    fw = (ALPHA * t3 + (1.0 - ALPHA) * (1.0 - t3)) * (ompt * ompt)
    lcls = jnp.sum(ce * fw) * (CLS_W / float(B * G))

    t6 = gath3[:, 6:7, :]
    nt = (gath3[:, 0:1, :], gath3[:, 1:2, :],
          jnp.log(jnp.clip(gath3[:, 3:4, :], 1e-6)),
          jnp.log(jnp.clip(gath3[:, 4:5, :], 1e-6)),
          gath3[:, 2:3, :],
          jnp.log(jnp.clip(gath3[:, 5:6, :], 1e-6)),
          jnp.sin(t6), jnp.cos(t6),
          gath3[:, 7:8, :], gath3[:, 8:9, :])
    finite = jnp.isfinite(nt[0])
    for d in range(1, 10):
        finite = jnp.logical_and(finite, jnp.isfinite(nt[d]))
    bw3 = pos3.astype(jnp.float32) * finite.astype(jnp.float32)
    acc = jnp.zeros((B, 1, Q), jnp.float32)
    for d in range(10):
        sd = jnp.where(finite, nt[d], 0.0)
        acc = acc + jnp.abs(pb3[:, d:d + 1, :] - sd) * CODE_W[d]
    lbb = jnp.sum(acc * bw3) * (BBOX_W / float(B * G))

    out_ref[0, 0] = lcls + lbb


_CALL = pl.pallas_call(
    _body,
    out_specs=pl.BlockSpec(memory_space=pltpu.SMEM),
    out_shape=jax.ShapeDtypeStruct((1, 1), jnp.float32),
    scratch_shapes=[
        pltpu.VMEM((B, G, Q), jnp.float32),
        pltpu.VMEM((B, G, 1), jnp.float32),
        pltpu.VMEM((B, G, 1), jnp.int32),
        pltpu.VMEM((B, 1, Q), jnp.float32),
        pltpu.VMEM((B, 1, Q), jnp.int32),
        pltpu.VMEM((B, 10, Q), jnp.float32),
    ],
)


def kernel(pred_bboxes, pred_logits, gt_bboxes_3d, gt_labels_3d):
    pb_t = jnp.transpose(pred_bboxes, (0, 2, 1))
    lg_t = jnp.transpose(pred_logits, (0, 2, 1))
    glc = gt_labels_3d[..., None]
    aug = jnp.concatenate(
        [jnp.transpose(gt_bboxes_3d, (0, 2, 1)),
         gt_labels_3d[:, None, :].astype(jnp.float32)], axis=1)
    out = _CALL(pb_t, lg_t, gt_bboxes_3d, glc, aug)
    return out[0, 0]


# speedup vs baseline: 30.1026x; 1.1889x over previous
import jax
import jax.numpy as jnp
from jax import lax
from jax.experimental import pallas as pl
from jax.experimental.pallas import tpu as pltpu

B, Q, G, C = 4, 900, 64, 10
K = 8
ALPHA, GAMMA = 0.25, 2.0
CLS_W, BBOX_W = 2.0, 0.25
EPS = 1e-12
INF = 1e9
BIGI = 2**30


def _body(pb_ref, lg_ref, gtn_ref, gtt_ref, glc_ref, glr_ref, out_ref,
          m_ref, v_ref, f_ref, qcol_ref, av_ref):
    for b in range(B):
        lg = lg_ref[b]
        p = 1.0 / (1.0 + jnp.exp(-lg))
        omp = 1.0 - p
        posc = -jnp.log(p + EPS) * ALPHA * (omp * omp)
        negc = -jnp.log(omp + EPS) * (1.0 - ALPHA) * (p * p)
        diff = (posc - negc) * CLS_W
        glr = glr_ref[b]
        ic_col = lax.broadcasted_iota(jnp.int32, (C, 1), 0)
        oh_cg = (ic_col == glr).astype(jnp.float32)
        c_cls = jnp.dot(diff, oh_cg, preferred_element_type=jnp.float32)
        gtt = gtt_ref[b]
        n0 = gtt[0:1, :]
        n1 = gtt[1:2, :]
        n2 = jnp.log(jnp.clip(gtt[3:4, :], 1e-6))
        n3 = jnp.log(jnp.clip(gtt[4:5, :], 1e-6))
        n4 = gtt[2:3, :]
        n5 = jnp.log(jnp.clip(gtt[5:6, :], 1e-6))
        n6 = jnp.sin(gtt[6:7, :])
        n7 = jnp.cos(gtt[6:7, :])
        pb = pb_ref[b]
        reg = jnp.abs(pb[:, 0:1] - n0)
        for d, nd in enumerate((n1, n2, n3, n4, n5, n6, n7), start=1):
            reg = reg + jnp.abs(pb[:, d:d + 1] - nd)
        m_ref[b] = c_cls + BBOX_W * reg

    iq3 = lax.broadcasted_iota(jnp.int32, (B, Q, G), 1)
    for k in range(K):
        m3 = m_ref[...]
        cmn = jnp.min(m3, axis=1, keepdims=True)
        fq = jnp.min(jnp.where(m3 == cmn, iq3, BIGI), axis=1, keepdims=True)
        v_ref[:, k:k + 1, :] = cmn
        f_ref[:, k:k + 1, :] = fq
        m_ref[...] = jnp.where(iq3 == fq, INF, m3)

    qcol_ref[...] = jnp.zeros((B, 1, G), jnp.int32)
    av_ref[...] = jnp.ones((B, 1, G), jnp.float32)
    ig_row = lax.broadcasted_iota(jnp.int32, (1, 1, G), 2)
    iqg = lax.broadcasted_iota(jnp.int32, (Q, G), 0)

    def step(_, carry):
        v = v_ref[...]
        fv = f_ref[...]
        cmk = jnp.min(v, axis=1, keepdims=True)
        fqk = jnp.min(jnp.where(v == cmk, fv, BIGI), axis=1, keepdims=True)
        keyrow = fqk * G + ig_row
        mv = jnp.min(cmk, axis=2, keepdims=True)
        key = jnp.min(jnp.where(cmk == mv, keyrow, BIGI), axis=2, keepdims=True)
        qs = key // G
        gs = key - qs * G
        colhit = ig_row == gs
        v_ref[...] = jnp.where(jnp.logical_or(fv == qs, colhit), INF, v)
        qcol_ref[...] = qcol_ref[...] + jnp.where(colhit, qs, 0)
        av_ref[...] = jnp.where(colhit, 0.0, av_ref[...])
        cmk2 = jnp.min(v_ref[...], axis=1, keepdims=True)
        exh = jnp.logical_and(cmk2 > 1e8, av_ref[...] > 0.5)
        for b in range(B):
            @pl.when(jnp.max(exh[b].astype(jnp.float32)) > 0.5)
            def _refill(b=b):
                qcol_b = qcol_ref[b]
                avb = av_ref[b]
                consumed = jnp.logical_and(iqg == qcol_b, avb < 0.5)
                rowdone = jnp.any(consumed, axis=1, keepdims=True)
                mm = jnp.where(rowdone, INF, m_ref[b])
                cmn = jnp.min(mm, axis=0, keepdims=True)
                fn = jnp.min(jnp.where(mm == cmn, iqg, BIGI),
                             axis=0, keepdims=True)
                ik_col = lax.broadcasted_iota(jnp.int32, (K, 1), 0)
                exh_b = exh[b]
                slot0 = jnp.logical_and(ik_col == 0, exh_b)
                v_ref[b] = jnp.where(slot0, cmn, v_ref[b])
                f_ref[b] = jnp.where(slot0, fn, f_ref[b])
        return carry

    lax.fori_loop(0, G, step, 0)

    ic_row = lax.broadcasted_iota(jnp.int32, (1, C), 1)
    cw_row = jnp.where(ic_row < 8, 1.0, 0.2)
    total = jnp.zeros((), jnp.float32)
    for b in range(B):
        gtn = gtn_ref[b]
        glc = glc_ref[b].astype(jnp.float32)
        c0 = gtn[:, 0:1]
        c1 = gtn[:, 1:2]
        c2 = jnp.log(jnp.clip(gtn[:, 3:4], 1e-6))
        c3 = jnp.log(jnp.clip(gtn[:, 4:5], 1e-6))
        c4 = gtn[:, 2:3]
        c5 = jnp.log(jnp.clip(gtn[:, 5:6], 1e-6))
        c6 = jnp.sin(gtn[:, 6:7])
        c7 = jnp.cos(gtn[:, 6:7])
        c8 = gtn[:, 7:8]
        c9 = gtn[:, 8:9]
        ones = jnp.ones((G, 1), jnp.float32)
        naug = jnp.concatenate(
            [c0, c1, c2, c3, c4, c5, c6, c7, c8, c9, glc, ones], axis=1)
        fing = jnp.all(jnp.isfinite(naug[:, 0:10]), axis=1, keepdims=True)
        naug = jnp.concatenate([naug, fing.astype(jnp.float32)], axis=1)
        oh = (iqg == qcol_ref[b]).astype(jnp.float32)
        gath = jnp.dot(oh, naug, preferred_element_type=jnp.float32)
        posq = gath[:, 11:12]
        bwq = gath[:, 12:13]
        labels = jnp.where(posq > 0.5, gath[:, 10:11], float(C))
        x = lg_ref[b]
        p = 1.0 / (1.0 + jnp.exp(-x))
        t = (ic_row == labels.astype(jnp.int32)).astype(jnp.float32)
        ce = jnp.maximum(x, 0.0) - x * t + jnp.log(1.0 + jnp.exp(-jnp.abs(x)))
        pt = p * t + (1.0 - p) * (1.0 - t)
        ompt = 1.0 - pt
        fw = (ALPHA * t + (1.0 - ALPHA) * (1.0 - t)) * (ompt * ompt)
        lcls = jnp.sum(ce * fw) * (CLS_W / float(B * G))
        sd = gath[:, 0:10]
        sd = jnp.where(jnp.isfinite(sd), sd, 0.0)
        lbb = jnp.sum(jnp.abs(pb_ref[b] - sd) * cw_row * bwq) \
            * (BBOX_W / float(B * G))
        total = total + lcls + lbb

    out_ref[0, 0] = total


_CALL = pl.pallas_call(
    _body,
    out_specs=pl.BlockSpec(memory_space=pltpu.SMEM),
    out_shape=jax.ShapeDtypeStruct((1, 1), jnp.float32),
    scratch_shapes=[
        pltpu.VMEM((B, Q, G), jnp.float32),
        pltpu.VMEM((B, K, G), jnp.float32),
        pltpu.VMEM((B, K, G), jnp.int32),
        pltpu.VMEM((B, 1, G), jnp.int32),
        pltpu.VMEM((B, 1, G), jnp.float32),
    ],
)


def kernel(pred_bboxes, pred_logits, gt_bboxes_3d, gt_labels_3d):
    gt_t = jnp.transpose(gt_bboxes_3d, (0, 2, 1))
    glc = gt_labels_3d[..., None]
    glr = gt_labels_3d[:, None, :]
    out = _CALL(pred_bboxes, pred_logits, gt_bboxes_3d, gt_t, glc, glr)
    return out[0, 0]


# speedup vs baseline: 39.0202x; 1.2962x over previous
import jax
import jax.numpy as jnp
from jax import lax
from jax.experimental import pallas as pl
from jax.experimental.pallas import tpu as pltpu

B, Q, G, C = 4, 900, 64, 10
K = 8
ALPHA, GAMMA = 0.25, 2.0
CLS_W, BBOX_W = 2.0, 0.25
EPS = 1e-12
INF = 1e9
BIGI = 2**30


def _body(pb_ref, lg_ref, gtn_ref, gtt_ref, glc_ref, glr_ref, out_ref,
          m_ref, v_ref, f_ref, qcol_ref, av_ref, cm_ref, fq_ref):
    for b in range(B):
        lg = lg_ref[b]
        p = 1.0 / (1.0 + jnp.exp(-lg))
        omp = 1.0 - p
        posc = -jnp.log(p + EPS) * ALPHA * (omp * omp)
        negc = -jnp.log(omp + EPS) * (1.0 - ALPHA) * (p * p)
        diff = (posc - negc) * CLS_W
        glr = glr_ref[b]
        ic_col = lax.broadcasted_iota(jnp.int32, (C, 1), 0)
        oh_cg = (ic_col == glr).astype(jnp.float32)
        c_cls = jnp.dot(diff, oh_cg, preferred_element_type=jnp.float32)
        gtt = gtt_ref[b]
        n0 = gtt[0:1, :]
        n1 = gtt[1:2, :]
        n2 = jnp.log(jnp.clip(gtt[3:4, :], 1e-6))
        n3 = jnp.log(jnp.clip(gtt[4:5, :], 1e-6))
        n4 = gtt[2:3, :]
        n5 = jnp.log(jnp.clip(gtt[5:6, :], 1e-6))
        n6 = jnp.sin(gtt[6:7, :])
        n7 = jnp.cos(gtt[6:7, :])
        pb = pb_ref[b]
        reg = jnp.abs(pb[:, 0:1] - n0)
        for d, nd in enumerate((n1, n2, n3, n4, n5, n6, n7), start=1):
            reg = reg + jnp.abs(pb[:, d:d + 1] - nd)
        m_ref[b] = c_cls + BBOX_W * reg

    iq3 = lax.broadcasted_iota(jnp.int32, (B, Q, G), 1)
    for k in range(K):
        m3 = m_ref[...]
        cmn = jnp.min(m3, axis=1, keepdims=True)
        fq = jnp.min(jnp.where(m3 == cmn, iq3, BIGI), axis=1, keepdims=True)
        v_ref[:, k:k + 1, :] = cmn
        f_ref[:, k:k + 1, :] = fq
        m_ref[...] = jnp.where(iq3 == fq, INF, m3)

    qcol_ref[...] = jnp.zeros((B, 1, G), jnp.int32)
    av_ref[...] = jnp.ones((B, 1, G), jnp.float32)
    ig_row = lax.broadcasted_iota(jnp.int32, (1, 1, G), 2)
    iqg = lax.broadcasted_iota(jnp.int32, (Q, G), 0)

    v0 = v_ref[...]
    f0 = f_ref[...]
    cm_ref[...] = jnp.min(v0, axis=1, keepdims=True)
    fq_ref[...] = jnp.min(jnp.where(v0 == cm_ref[...], f0, BIGI),
                          axis=1, keepdims=True)

    def step(_, carry):
        cmk = cm_ref[...]
        fqk = fq_ref[...]
        keyrow = fqk * G + ig_row
        mv = jnp.min(cmk, axis=2, keepdims=True)
        key = jnp.min(jnp.where(cmk == mv, keyrow, BIGI), axis=2, keepdims=True)
        qs = lax.shift_right_logical(key, 6)
        gs = jnp.bitwise_and(key, G - 1)
        colhit = ig_row == gs
        v = v_ref[...]
        fv = f_ref[...]
        vnew = jnp.where(jnp.logical_or(fv == qs, colhit), INF, v)
        v_ref[...] = vnew
        qcol_ref[...] = qcol_ref[...] + jnp.where(colhit, qs, 0)
        av = jnp.where(colhit, 0.0, av_ref[...])
        av_ref[...] = av
        cmk2 = jnp.min(vnew, axis=1, keepdims=True)
        fqk2 = jnp.min(jnp.where(vnew == cmk2, fv, BIGI), axis=1, keepdims=True)
        cm_ref[...] = cmk2
        fq_ref[...] = fqk2
        exh = jnp.logical_and(cmk2 > 1e8, av > 0.5)

        @pl.when(jnp.max(exh.astype(jnp.float32)) > 0.5)
        def _any_refill():
            for b in range(B):
                @pl.when(jnp.max(exh[b].astype(jnp.float32)) > 0.5)
                def _refill(b=b):
                    qcol_b = qcol_ref[b]
                    avb = av_ref[b]
                    consumed = jnp.logical_and(iqg == qcol_b, avb < 0.5)
                    rowdone = jnp.any(consumed, axis=1, keepdims=True)
                    mm = jnp.where(rowdone, INF, m_ref[b])
                    cmn = jnp.min(mm, axis=0, keepdims=True)
                    fn = jnp.min(jnp.where(mm == cmn, iqg, BIGI),
                                 axis=0, keepdims=True)
                    ik_col = lax.broadcasted_iota(jnp.int32, (K, 1), 0)
                    exh_b = exh[b]
                    slot0 = jnp.logical_and(ik_col == 0, exh_b)
                    v_ref[b] = jnp.where(slot0, cmn, v_ref[b])
                    f_ref[b] = jnp.where(slot0, fn, f_ref[b])
                    cm_ref[b] = jnp.where(exh_b, cmn, cm_ref[b])
                    fq_ref[b] = jnp.where(exh_b, fn, fq_ref[b])
        return carry

    lax.fori_loop(0, G, step, 0)

    ic_row = lax.broadcasted_iota(jnp.int32, (1, C), 1)
    cw_row = jnp.where(ic_row < 8, 1.0, 0.2)
    total = jnp.zeros((), jnp.float32)
    for b in range(B):
        gtn = gtn_ref[b]
        glc = glc_ref[b].astype(jnp.float32)
        c0 = gtn[:, 0:1]
        c1 = gtn[:, 1:2]
        c2 = jnp.log(jnp.clip(gtn[:, 3:4], 1e-6))
        c3 = jnp.log(jnp.clip(gtn[:, 4:5], 1e-6))
        c4 = gtn[:, 2:3]
        c5 = jnp.log(jnp.clip(gtn[:, 5:6], 1e-6))
        c6 = jnp.sin(gtn[:, 6:7])
        c7 = jnp.cos(gtn[:, 6:7])
        c8 = gtn[:, 7:8]
        c9 = gtn[:, 8:9]
        ones = jnp.ones((G, 1), jnp.float32)
        naug = jnp.concatenate(
            [c0, c1, c2, c3, c4, c5, c6, c7, c8, c9, glc, ones], axis=1)
        fing = jnp.all(jnp.isfinite(naug[:, 0:10]), axis=1, keepdims=True)
        naug = jnp.concatenate([naug, fing.astype(jnp.float32)], axis=1)
        oh = (iqg == qcol_ref[b]).astype(jnp.float32)
        gath = jnp.dot(oh, naug, preferred_element_type=jnp.float32)
        posq = gath[:, 11:12]
        bwq = gath[:, 12:13]
        labels = jnp.where(posq > 0.5, gath[:, 10:11], float(C))
        x = lg_ref[b]
        p = 1.0 / (1.0 + jnp.exp(-x))
        t = (ic_row == labels.astype(jnp.int32)).astype(jnp.float32)
        ce = jnp.maximum(x, 0.0) - x * t + jnp.log(1.0 + jnp.exp(-jnp.abs(x)))
        pt = p * t + (1.0 - p) * (1.0 - t)
        ompt = 1.0 - pt
        fw = (ALPHA * t + (1.0 - ALPHA) * (1.0 - t)) * (ompt * ompt)
        lcls = jnp.sum(ce * fw) * (CLS_W / float(B * G))
        sd = gath[:, 0:10]
        sd = jnp.where(jnp.isfinite(sd), sd, 0.0)
        lbb = jnp.sum(jnp.abs(pb_ref[b] - sd) * cw_row * bwq) \
            * (BBOX_W / float(B * G))
        total = total + lcls + lbb

    out_ref[0, 0] = total


_CALL = pl.pallas_call(
    _body,
    out_specs=pl.BlockSpec(memory_space=pltpu.SMEM),
    out_shape=jax.ShapeDtypeStruct((1, 1), jnp.float32),
    scratch_shapes=[
        pltpu.VMEM((B, Q, G), jnp.float32),
        pltpu.VMEM((B, K, G), jnp.float32),
        pltpu.VMEM((B, K, G), jnp.int32),
        pltpu.VMEM((B, 1, G), jnp.int32),
        pltpu.VMEM((B, 1, G), jnp.float32),
        pltpu.VMEM((B, 1, G), jnp.float32),
        pltpu.VMEM((B, 1, G), jnp.int32),
    ],
)


def kernel(pred_bboxes, pred_logits, gt_bboxes_3d, gt_labels_3d):
    gt_t = jnp.transpose(gt_bboxes_3d, (0, 2, 1))
    glc = gt_labels_3d[..., None]
    glr = gt_labels_3d[:, None, :]
    out = _CALL(pred_bboxes, pred_logits, gt_bboxes_3d, gt_t, glc, glr)
    return out[0, 0]


# speedup vs baseline: 45.6018x; 1.1687x over previous
import jax
import jax.numpy as jnp
from jax import lax
from jax.experimental import pallas as pl
from jax.experimental.pallas import tpu as pltpu

B, Q, G, C = 4, 900, 64, 10
K = 16
ALPHA, GAMMA = 0.25, 2.0
CLS_W, BBOX_W = 2.0, 0.25
EPS = 1e-12
INF = 1e9
BIGI = 2**30


def _body(pb_ref, lg_ref, gtn_ref, gtt_ref, glc_ref, glr_ref, out_ref,
          m_ref, v_ref, f_ref, qcol_ref, cm_ref, fq_ref, wm_ref,
          rmq_ref, cmsk_ref):

    def build_cost(b):
        lg = lg_ref[b]
        p = 1.0 / (1.0 + jnp.exp(-lg))
        omp = 1.0 - p
        posc = -jnp.log(p + EPS) * ALPHA * (omp * omp)
        negc = -jnp.log(omp + EPS) * (1.0 - ALPHA) * (p * p)
        diff = (posc - negc) * CLS_W
        glr = glr_ref[b]
        ic_col = lax.broadcasted_iota(jnp.int32, (C, 1), 0)
        oh_cg = (ic_col == glr).astype(jnp.float32)
        c_cls = jnp.dot(diff, oh_cg, preferred_element_type=jnp.float32)
        gtt = gtt_ref[b]
        n0 = gtt[0:1, :]
        n1 = gtt[1:2, :]
        n2 = jnp.log(jnp.clip(gtt[3:4, :], 1e-6))
        n3 = jnp.log(jnp.clip(gtt[4:5, :], 1e-6))
        n4 = gtt[2:3, :]
        n5 = jnp.log(jnp.clip(gtt[5:6, :], 1e-6))
        n6 = jnp.sin(gtt[6:7, :])
        n7 = jnp.cos(gtt[6:7, :])
        pb = pb_ref[b]
        reg = jnp.abs(pb[:, 0:1] - n0)
        for d, nd in enumerate((n1, n2, n3, n4, n5, n6, n7), start=1):
            reg = reg + jnp.abs(pb[:, d:d + 1] - nd)
        return c_cls + BBOX_W * reg

    for b in range(B):
        m_ref[b] = build_cost(b)

    iq3 = lax.broadcasted_iota(jnp.int32, (B, Q, G), 1)
    for k in range(K):
        m3 = m_ref[...]
        cmn = jnp.min(m3, axis=1, keepdims=True)
        fq = jnp.min(jnp.where(m3 == cmn, iq3, BIGI), axis=1, keepdims=True)
        v_ref[:, k:k + 1, :] = cmn
        f_ref[:, k:k + 1, :] = fq
        m_ref[...] = jnp.where(iq3 == fq, INF, m3)

    qcol_ref[...] = jnp.zeros((B, 1, G), jnp.int32)
    wm_ref[...] = jnp.zeros((B, 1, 1), jnp.float32)
    ig_row = lax.broadcasted_iota(jnp.int32, (1, 1, G), 2)
    iqg = lax.broadcasted_iota(jnp.int32, (Q, G), 0)

    v0 = v_ref[...]
    f0 = f_ref[...]
    cm_ref[...] = jnp.min(v0, axis=1, keepdims=True)
    fq_ref[...] = jnp.min(jnp.where(v0 == cm_ref[...], f0, BIGI),
                          axis=1, keepdims=True)

    def step(_, carry):
        cmk = cm_ref[...]
        fqk = fq_ref[...]
        keyrow = fqk * G + ig_row
        mv = jnp.min(cmk, axis=2, keepdims=True)
        key = jnp.min(jnp.where(cmk == mv, keyrow, BIGI), axis=2, keepdims=True)
        qs = lax.shift_right_logical(key, 6)
        gs = jnp.bitwise_and(key, G - 1)
        colhit = ig_row == gs
        v = v_ref[...]
        fv = f_ref[...]
        vnew = jnp.where(jnp.logical_or(fv == qs, colhit), INF, v)
        v_ref[...] = vnew
        qcol_ref[...] = qcol_ref[...] + jnp.where(colhit, qs, 0)
        cm_ref[...] = jnp.min(vnew, axis=1, keepdims=True)
        fq_ref[...] = jnp.min(jnp.where(vnew == cm_ref[...], fv, BIGI),
                              axis=1, keepdims=True)
        wm_ref[...] = jnp.maximum(wm_ref[...], mv)
        return carry

    lax.fori_loop(0, G, step, 0)

    @pl.when(jnp.max(wm_ref[...]) > 1e8)
    def _redo():
        for b in range(B):
            m_ref[b] = build_cost(b)
        qcol_ref[...] = jnp.zeros((B, 1, G), jnp.int32)
        rmq_ref[...] = jnp.zeros((B, Q, 1), jnp.float32)
        cmsk_ref[...] = jnp.zeros((B, 1, G), jnp.float32)
        iq3q = lax.broadcasted_iota(jnp.int32, (B, Q, 1), 1)

        def rstep(_, carry):
            m3 = m_ref[...] + rmq_ref[...] + cmsk_ref[...]
            cmn = jnp.min(m3, axis=1, keepdims=True)
            fq = jnp.min(jnp.where(m3 == cmn, iq3, BIGI), axis=1, keepdims=True)
            keyrow = fq * G + ig_row
            mv = jnp.min(cmn, axis=2, keepdims=True)
            key = jnp.min(jnp.where(cmn == mv, keyrow, BIGI),
                          axis=2, keepdims=True)
            qs = lax.shift_right_logical(key, 6)
            gs = jnp.bitwise_and(key, G - 1)
            colhit = ig_row == gs
            qcol_ref[...] = qcol_ref[...] + jnp.where(colhit, qs, 0)
            cmsk_ref[...] = cmsk_ref[...] + jnp.where(colhit, INF, 0.0)
            rmq_ref[...] = rmq_ref[...] + jnp.where(iq3q == qs, INF, 0.0)
            return carry

        lax.fori_loop(0, G, rstep, 0)

    ic_row = lax.broadcasted_iota(jnp.int32, (1, C), 1)
    cw_row = jnp.where(ic_row < 8, 1.0, 0.2)
    total = jnp.zeros((), jnp.float32)
    for b in range(B):
        gtn = gtn_ref[b]
        glc = glc_ref[b].astype(jnp.float32)
        c0 = gtn[:, 0:1]
        c1 = gtn[:, 1:2]
        c2 = jnp.log(jnp.clip(gtn[:, 3:4], 1e-6))
        c3 = jnp.log(jnp.clip(gtn[:, 4:5], 1e-6))
        c4 = gtn[:, 2:3]
        c5 = jnp.log(jnp.clip(gtn[:, 5:6], 1e-6))
        c6 = jnp.sin(gtn[:, 6:7])
        c7 = jnp.cos(gtn[:, 6:7])
        c8 = gtn[:, 7:8]
        c9 = gtn[:, 8:9]
        ones = jnp.ones((G, 1), jnp.float32)
        naug = jnp.concatenate(
            [c0, c1, c2, c3, c4, c5, c6, c7, c8, c9, glc, ones], axis=1)
        fing = jnp.all(jnp.isfinite(naug[:, 0:10]), axis=1, keepdims=True)
        naug = jnp.concatenate([naug, fing.astype(jnp.float32)], axis=1)
        oh = (iqg == qcol_ref[b]).astype(jnp.float32)
        gath = jnp.dot(oh, naug, preferred_element_type=jnp.float32)
        posq = gath[:, 11:12]
        bwq = gath[:, 12:13]
        labels = jnp.where(posq > 0.5, gath[:, 10:11], float(C))
        x = lg_ref[b]
        p = 1.0 / (1.0 + jnp.exp(-x))
        t = (ic_row == labels.astype(jnp.int32)).astype(jnp.float32)
        ce = jnp.maximum(x, 0.0) - x * t + jnp.log(1.0 + jnp.exp(-jnp.abs(x)))
        pt = p * t + (1.0 - p) * (1.0 - t)
        ompt = 1.0 - pt
        fw = (ALPHA * t + (1.0 - ALPHA) * (1.0 - t)) * (ompt * ompt)
        lcls = jnp.sum(ce * fw) * (CLS_W / float(B * G))
        sd = gath[:, 0:10]
        sd = jnp.where(jnp.isfinite(sd), sd, 0.0)
        lbb = jnp.sum(jnp.abs(pb_ref[b] - sd) * cw_row * bwq) \
            * (BBOX_W / float(B * G))
        total = total + lcls + lbb

    out_ref[0, 0] = total


_CALL = pl.pallas_call(
    _body,
    out_specs=pl.BlockSpec(memory_space=pltpu.SMEM),
    out_shape=jax.ShapeDtypeStruct((1, 1), jnp.float32),
    scratch_shapes=[
        pltpu.VMEM((B, Q, G), jnp.float32),
        pltpu.VMEM((B, K, G), jnp.float32),
        pltpu.VMEM((B, K, G), jnp.int32),
        pltpu.VMEM((B, 1, G), jnp.int32),
        pltpu.VMEM((B, 1, G), jnp.float32),
        pltpu.VMEM((B, 1, G), jnp.int32),
        pltpu.VMEM((B, 1, 1), jnp.float32),
        pltpu.VMEM((B, Q, 1), jnp.float32),
        pltpu.VMEM((B, 1, G), jnp.float32),
    ],
)


def kernel(pred_bboxes, pred_logits, gt_bboxes_3d, gt_labels_3d):
    gt_t = jnp.transpose(gt_bboxes_3d, (0, 2, 1))
    glc = gt_labels_3d[..., None]
    glr = gt_labels_3d[:, None, :]
    out = _CALL(pred_bboxes, pred_logits, gt_bboxes_3d, gt_t, glc, glr)
    return out[0, 0]
